# baseline (speedup 1.0000x reference)
import jax
import jax.numpy as jnp
from jax import lax
from jax.experimental import pallas as pl
from jax.experimental.pallas import tpu as pltpu
from jax.experimental.pallas import tpu_sc as plsc

VOCAB = 1000000
EMBED_DIM = 32
BATCH = 4096
HIST = 200

NUM_CORES = 2
NUM_SUBCORES = 16
NW = NUM_CORES * NUM_SUBCORES
LANES = 16
NB = 128
NGRP = NB // LANES


def _gather_body(e2_hbm, t128_hbm, o3_hbm,
                 ivmem, idx4a, idx4b, bcola, bcolb, ga, gb, oa, ob,
                 sga, sgb, soa, sob):
    w = lax.axis_index("s") * NUM_CORES + lax.axis_index("c")
    pltpu.sync_copy(e2_hbm.at[:, pl.ds(w * NB, NB)], ivmem)

    idx4 = [idx4a, idx4b]
    bcol = [bcola, bcolb]
    gbuf = [ga, gb]
    obuf = [oa, ob]
    sg = [sga, sgb]
    so = [soa, sob]

    iota = lax.iota(jnp.int32, LANES)

    def prep(h, b):
        for gi in range(NGRP):
            v = ivmem[h, pl.ds(LANES * gi, LANES)]
            idx4[b][pl.ds(LANES * gi, LANES)] = v >> 2
            bcol[b][pl.ds(LANES * gi, LANES)] = (v & 3) << 5

    def start_gather(b):
        return pltpu.async_copy(t128_hbm.at[idx4[b]], gbuf[b], sg[b])

    def transpose(b):
        for gi in range(NGRP):
            rows = iota + (LANES * gi)
            c0 = bcol[b][pl.ds(LANES * gi, LANES)]
            for d in range(EMBED_DIM):
                obuf[b][d, pl.ds(LANES * gi, LANES)] = plsc.load_gather(
                    gbuf[b], [rows, c0 + d])

    def start_store(h, b):
        return pltpu.async_copy(
            obuf[b], o3_hbm.at[h, :, pl.ds(w * NB, NB)], so[b])

    def wait_gather(b):
        pltpu.make_async_copy(t128_hbm.at[idx4[b]], gbuf[b], sg[b]).wait()

    def wait_store(h, b):
        pltpu.make_async_copy(
            obuf[b], o3_hbm.at[h, :, pl.ds(w * NB, NB)], so[b]).wait()

    def step(h, b, prep_next, wait_out):
        if prep_next:
            prep(h + 1, 1 - b)
            start_gather(1 - b)
        wait_gather(b)
        if wait_out:
            wait_store(h, b)
        transpose(b)
        start_store(h, b)

    prep(0, 0)
    start_gather(0)
    step(0, 0, True, False)
    step(1, 1, True, False)

    def body(j, carry):
        step(2 * j, 0, True, True)
        step(2 * j + 1, 1, True, True)
        return carry

    lax.fori_loop(1, HIST // 2 - 1, body, 0)

    step(HIST - 2, 0, True, True)
    step(HIST - 1, 1, False, True)
    wait_store(0, 0)
    wait_store(0, 1)


@jax.jit
def kernel(to_embed, table):
    e2 = to_embed.T
    t128 = table.reshape(VOCAB // 4, 128)
    mesh = plsc.VectorSubcoreMesh(core_axis_name="c", subcore_axis_name="s")
    o3 = pl.kernel(
        _gather_body,
        out_type=jax.ShapeDtypeStruct((HIST, EMBED_DIM, BATCH), jnp.float32),
        mesh=mesh,
        scratch_types=(
            [pltpu.VMEM((HIST, NB), jnp.int32)]
            + [pltpu.VMEM((NB,), jnp.int32) for _ in range(4)]
            + [pltpu.VMEM((NB, 128), jnp.float32) for _ in range(2)]
            + [pltpu.VMEM((EMBED_DIM, NB), jnp.float32) for _ in range(2)]
            + [pltpu.SemaphoreType.DMA for _ in range(4)]
        ),
        compiler_params=pltpu.CompilerParams(
            use_tc_tiling_on_sc=True, needs_layout_passes=False),
    )(e2, t128)
    return o3.transpose(2, 0, 1)


# speedup vs baseline: 1.4631x; 1.4631x over previous
import jax
import jax.numpy as jnp
from jax import lax
from jax.experimental import pallas as pl
from jax.experimental.pallas import tpu as pltpu
from jax.experimental.pallas import tpu_sc as plsc

VOCAB = 1000000
EMBED_DIM = 32
BATCH = 4096
HIST = 200

NUM_CORES = 2
NUM_SUBCORES = 16
NW = NUM_CORES * NUM_SUBCORES
LANES = 16
NB = 128
NGRP = NB // LANES


def _gather_body(e2_hbm, t128_hbm, o3_hbm,
                 ivmem, idx4a, idx4b, bcola, bcolb, ga, gb, oa, ob,
                 sga, sgb, soa, sob):
    w = lax.axis_index("s") * NUM_CORES + lax.axis_index("c")
    pltpu.sync_copy(e2_hbm.at[:, pl.ds(w * NB, NB)], ivmem)

    idx4 = [idx4a, idx4b]
    bcol = [bcola, bcolb]
    gbuf = [ga, gb]
    obuf = [oa, ob]
    sg = [sga, sgb]
    so = [soa, sob]

    iota = lax.iota(jnp.int32, LANES)

    def prep(h, b):
        for gi in range(NGRP):
            v = ivmem[h, pl.ds(LANES * gi, LANES)]
            idx4[b][pl.ds(LANES * gi, LANES)] = v >> 2
            bcol[b][pl.ds(LANES * gi, LANES)] = (v & 3) << 5

    def start_gather(b):
        return pltpu.async_copy(t128_hbm.at[idx4[b]], gbuf[b], sg[b])

    def transpose(b):
        def grp(gi, carry):
            rows = iota + LANES * gi
            c0 = bcol[b][pl.ds(LANES * gi, LANES)]
            for d0 in range(EMBED_DIM):
                dvec = (iota + d0) & (EMBED_DIM - 1)
                vals = plsc.load_gather(gbuf[b], [rows, c0 + dvec])
                plsc.store_scatter(obuf[b], [dvec, rows], vals)
            return carry

        lax.fori_loop(0, NGRP, grp, 0)

    def start_store(h, b):
        return pltpu.async_copy(
            obuf[b], o3_hbm.at[h, :, pl.ds(w * NB, NB)], so[b])

    def wait_gather(b):
        pltpu.make_async_copy(t128_hbm.at[idx4[b]], gbuf[b], sg[b]).wait()

    def wait_store(h, b):
        pltpu.make_async_copy(
            obuf[b], o3_hbm.at[h, :, pl.ds(w * NB, NB)], so[b]).wait()

    def step(h, b, prep_next, wait_out):
        if prep_next:
            prep(h + 1, 1 - b)
            start_gather(1 - b)
        wait_gather(b)
        if wait_out:
            wait_store(h, b)
        transpose(b)
        start_store(h, b)

    prep(0, 0)
    start_gather(0)
    step(0, 0, True, False)
    step(1, 1, True, False)

    def body(j, carry):
        step(2 * j, 0, True, True)
        step(2 * j + 1, 1, True, True)
        return carry

    lax.fori_loop(1, HIST // 2 - 1, body, 0)

    step(HIST - 2, 0, True, True)
    step(HIST - 1, 1, False, True)
    wait_store(0, 0)
    wait_store(0, 1)


@jax.jit
def kernel(to_embed, table):
    e2 = to_embed.T
    t128 = table.reshape(VOCAB // 4, 128)
    mesh = plsc.VectorSubcoreMesh(core_axis_name="c", subcore_axis_name="s")
    o3 = pl.kernel(
        _gather_body,
        out_type=jax.ShapeDtypeStruct((HIST, EMBED_DIM, BATCH), jnp.float32),
        mesh=mesh,
        scratch_types=(
            [pltpu.VMEM((HIST, NB), jnp.int32)]
            + [pltpu.VMEM((NB,), jnp.int32) for _ in range(4)]
            + [pltpu.VMEM((NB, 128), jnp.float32) for _ in range(2)]
            + [pltpu.VMEM((EMBED_DIM, NB), jnp.float32) for _ in range(2)]
            + [pltpu.SemaphoreType.DMA for _ in range(4)]
        ),
        compiler_params=pltpu.CompilerParams(
            use_tc_tiling_on_sc=True, needs_layout_passes=False),
    )(e2, t128)
    return o3.transpose(2, 0, 1)


# speedup vs baseline: 1.5330x; 1.0477x over previous
import jax
import jax.numpy as jnp
from jax import lax
from jax.experimental import pallas as pl
from jax.experimental.pallas import tpu as pltpu
from jax.experimental.pallas import tpu_sc as plsc

VOCAB = 1000000
EMBED_DIM = 32
BATCH = 4096
HIST = 200

NUM_CORES = 2
NUM_SUBCORES = 16
NW = NUM_CORES * NUM_SUBCORES
LANES = 16
NB = 128
NGRP = NB // LANES


def _gather_body(e2_hbm, tpad_hbm, o3_hbm,
                 ivmem, ga, gb, oa, ob, sga, sgb, soa, sob):
    w = lax.axis_index("s") * NUM_CORES + lax.axis_index("c")
    pltpu.sync_copy(e2_hbm.at[:, pl.ds(w * NB, NB)], ivmem)

    gbuf = [ga, gb]
    obuf = [oa, ob]
    sg = [sga, sgb]
    so = [soa, sob]

    iota = lax.iota(jnp.int32, LANES)

    def start_gather(h, b):
        return pltpu.async_copy(tpad_hbm.at[ivmem.at[h]], gbuf[b], sg[b])

    def transpose(b):
        def grp(gi, carry):
            rows = iota + LANES * gi
            for d0 in range(EMBED_DIM):
                dvec = (iota + d0) & (EMBED_DIM - 1)
                vals = plsc.load_gather(gbuf[b], [rows, dvec])
                plsc.store_scatter(obuf[b], [dvec, rows], vals)
            return carry

        lax.fori_loop(0, NGRP, grp, 0)

    def start_store(h, b):
        return pltpu.async_copy(
            obuf[b], o3_hbm.at[h, :, pl.ds(w * NB, NB)], so[b])

    def wait_gather(h, b):
        pltpu.make_async_copy(tpad_hbm.at[ivmem.at[h]], gbuf[b], sg[b]).wait()

    def wait_store(h, b):
        pltpu.make_async_copy(
            obuf[b], o3_hbm.at[h, :, pl.ds(w * NB, NB)], so[b]).wait()

    def step(h, b, prefetch, wait_out):
        if prefetch:
            start_gather(h + 1, 1 - b)
        wait_gather(h, b)
        if wait_out:
            wait_store(h, b)
        transpose(b)
        start_store(h, b)

    start_gather(0, 0)
    step(0, 0, True, False)
    step(1, 1, True, False)

    def body(j, carry):
        step(2 * j, 0, True, True)
        step(2 * j + 1, 1, True, True)
        return carry

    lax.fori_loop(1, HIST // 2 - 1, body, 0)

    step(HIST - 2, 0, True, True)
    step(HIST - 1, 1, False, True)
    wait_store(0, 0)
    wait_store(0, 1)


@jax.jit
def kernel(to_embed, table):
    e2 = to_embed.T
    tpad = jnp.pad(table, ((0, 0), (0, 128 - EMBED_DIM)))
    mesh = plsc.VectorSubcoreMesh(core_axis_name="c", subcore_axis_name="s")
    o3 = pl.kernel(
        _gather_body,
        out_type=jax.ShapeDtypeStruct((HIST, EMBED_DIM, BATCH), jnp.float32),
        mesh=mesh,
        scratch_types=(
            [pltpu.VMEM((HIST, NB), jnp.int32)]
            + [pltpu.VMEM((NB, 128), jnp.float32) for _ in range(2)]
            + [pltpu.VMEM((EMBED_DIM, NB), jnp.float32) for _ in range(2)]
            + [pltpu.SemaphoreType.DMA for _ in range(4)]
        ),
        compiler_params=pltpu.CompilerParams(
            use_tc_tiling_on_sc=True, needs_layout_passes=False),
    )(e2, tpad)
    return o3.transpose(2, 0, 1)


# speedup vs baseline: 1.5462x; 1.0086x over previous
import jax
import jax.numpy as jnp
from jax import lax
from jax.experimental import pallas as pl
from jax.experimental.pallas import tpu as pltpu
from jax.experimental.pallas import tpu_sc as plsc

VOCAB = 1000000
EMBED_DIM = 32
BATCH = 4096
HIST = 200

NUM_CORES = 2
NUM_SUBCORES = 16
NW = NUM_CORES * NUM_SUBCORES
LANES = 16
NB = 128
NGRP = NB // LANES

NTILE = VOCAB // 128
NTAIL = VOCAB - NTILE * 128
TPW = NTILE // NW
TEXTRA = NTILE - TPW * NW
TROWS = VOCAB // 4 + 16


def _wid():
    return lax.axis_index("s") * NUM_CORES + lax.axis_index("c")


def _format_body(tt_hbm, ttail_hbm, t128_hbm,
                 ta, tb, oa, ob, si_a, si_b, so_a, so_b):
    w = _wid()
    base = w * TPW + jnp.minimum(w, TEXTRA)
    count = jnp.where(w < TEXTRA, TPW + 1, TPW)

    tbuf = [ta, tb]
    obuf = [oa, ob]
    si = [si_a, si_b]
    so = [so_a, so_b]
    iota = lax.iota(jnp.int32, LANES)

    def load(t, b, width):
        return pltpu.async_copy(
            tt_hbm.at[:, pl.ds(128 * t, width)],
            tbuf[b].at[:, pl.ds(0, width)], si[b])

    def transpose(b, ngrp):
        for gi in range(ngrp):
            vl = iota + LANES * gi
            rvec = vl >> 2
            cbase = (vl & 3) << 5
            for d0 in range(EMBED_DIM):
                dvec = (iota + d0) & (EMBED_DIM - 1)
                vals = plsc.load_gather(tbuf[b], [dvec, vl])
                plsc.store_scatter(obuf[b], [rvec, cbase + dvec], vals)

    def store(t, b, rows):
        return pltpu.async_copy(
            obuf[b].at[pl.ds(0, rows)],
            t128_hbm.at[pl.ds(32 * t, rows)], so[b])

    def step(i, b, prefetch_next):
        t = base + i
        pltpu.make_async_copy(
            tt_hbm.at[:, pl.ds(128 * t, 128)], tbuf[b], si[b]).wait()

        @pl.when(prefetch_next)
        def _():
            load(t + 1, 1 - b, 128)

        @pl.when(i >= 2)
        def _():
            pltpu.make_async_copy(
                obuf[b], t128_hbm.at[pl.ds(32 * t, 32)], so[b]).wait()

        transpose(b, NGRP)
        store(t, b, 32)

    load(base, 0, 128)

    def body(j, carry):
        @pl.when(2 * j < count)
        def _():
            step(2 * j, 0, (2 * j + 1) < count)

        @pl.when(2 * j + 1 < count)
        def _():
            step(2 * j + 1, 1, (2 * j + 2) < count)

        return carry

    lax.fori_loop(0, (TPW + 2) // 2, body, 0)

    pltpu.make_async_copy(
        obuf[0], t128_hbm.at[pl.ds(0, 32)], so[0]).wait()
    pltpu.make_async_copy(
        obuf[1], t128_hbm.at[pl.ds(0, 32)], so[1]).wait()

    @pl.when(w == NW - 1)
    def _():
        pltpu.sync_copy(ttail_hbm, ta)
        transpose(0, NTAIL // LANES)
        pltpu.sync_copy(oa.at[pl.ds(0, NTAIL // 4)],
                        t128_hbm.at[pl.ds(32 * NTILE, NTAIL // 4)])


def _gather_body(e2_hbm, t128_hbm, o3_hbm,
                 ivmem, idx4a, idx4b, bcola, bcolb,
                 ga, gb, oa, ob, sga, sgb, soa, sob):
    w = _wid()
    pltpu.sync_copy(e2_hbm.at[:, pl.ds(w * NB, NB)], ivmem)

    idx4 = [idx4a, idx4b]
    bcol = [bcola, bcolb]
    gbuf = [ga, gb]
    obuf = [oa, ob]
    sg = [sga, sgb]
    so = [soa, sob]
    iota = lax.iota(jnp.int32, LANES)

    def prep(h, b):
        for gi in range(NGRP):
            v = ivmem[h, pl.ds(LANES * gi, LANES)]
            idx4[b][pl.ds(LANES * gi, LANES)] = v >> 2
            bcol[b][pl.ds(LANES * gi, LANES)] = (v & 3) << 5

    def start_gather(b):
        return pltpu.async_copy(t128_hbm.at[idx4[b]], gbuf[b], sg[b])

    def transpose(b):
        def grp(gi, carry):
            rows = iota + LANES * gi
            c0 = bcol[b][pl.ds(LANES * gi, LANES)]
            for d0 in range(EMBED_DIM):
                dvec = (iota + d0) & (EMBED_DIM - 1)
                vals = plsc.load_gather(gbuf[b], [rows, c0 + dvec])
                plsc.store_scatter(obuf[b], [dvec, rows], vals)
            return carry

        lax.fori_loop(0, NGRP, grp, 0)

    def wait_gather(b):
        pltpu.make_async_copy(t128_hbm.at[idx4[b]], gbuf[b], sg[b]).wait()

    def wait_store(h, b):
        pltpu.make_async_copy(
            obuf[b], o3_hbm.at[h, :, pl.ds(w * NB, NB)], so[b]).wait()

    def step(h, b, prefetch, wait_out):
        if prefetch:
            prep(h + 1, 1 - b)
            start_gather(1 - b)
        wait_gather(b)
        if wait_out:
            wait_store(h, b)
        transpose(b)
        pltpu.async_copy(obuf[b], o3_hbm.at[h, :, pl.ds(w * NB, NB)], so[b])

    prep(0, 0)
    start_gather(0)
    step(0, 0, True, False)
    step(1, 1, True, False)

    def body(j, carry):
        step(2 * j, 0, True, True)
        step(2 * j + 1, 1, True, True)
        return carry

    lax.fori_loop(1, HIST // 2 - 1, body, 0)

    step(HIST - 2, 0, True, True)
    step(HIST - 1, 1, False, True)
    wait_store(0, 0)
    wait_store(0, 1)


@jax.jit
def kernel(to_embed, table):
    e2 = to_embed.T
    tt = table.T
    ttail = jnp.pad(table[NTILE * 128:], ((0, 128 - NTAIL), (0, 0))).T
    mesh = plsc.VectorSubcoreMesh(core_axis_name="c", subcore_axis_name="s")
    params = pltpu.CompilerParams(
        use_tc_tiling_on_sc=True, needs_layout_passes=False)

    t128 = pl.kernel(
        _format_body,
        out_type=jax.ShapeDtypeStruct((TROWS, 128), jnp.float32),
        mesh=mesh,
        scratch_types=(
            [pltpu.VMEM((EMBED_DIM, 128), jnp.float32) for _ in range(2)]
            + [pltpu.VMEM((32, 128), jnp.float32) for _ in range(2)]
            + [pltpu.SemaphoreType.DMA for _ in range(4)]
        ),
        compiler_params=params,
    )(tt, ttail)

    o3 = pl.kernel(
        _gather_body,
        out_type=jax.ShapeDtypeStruct((HIST, EMBED_DIM, BATCH), jnp.float32),
        mesh=mesh,
        scratch_types=(
            [pltpu.VMEM((HIST, NB), jnp.int32)]
            + [pltpu.VMEM((NB,), jnp.int32) for _ in range(4)]
            + [pltpu.VMEM((NB, 128), jnp.float32) for _ in range(2)]
            + [pltpu.VMEM((EMBED_DIM, NB), jnp.float32) for _ in range(2)]
            + [pltpu.SemaphoreType.DMA for _ in range(4)]
        ),
        compiler_params=params,
    )(e2, t128)
    return o3.transpose(2, 0, 1)
